# Initial kernel scaffold; baseline (speedup 1.0000x reference)
import functools

import numpy as np
import jax
import jax.numpy as jnp
from jax import lax
from jax.experimental import pallas as pl
from jax.experimental.pallas import tpu as pltpu
from jax.experimental.pallas import tpu_sc as plsc

_B, _H, _W = 2, 38, 38
_CIN = 512
_CMID = 512
_A = 9
_K = _H * _W
_N = _K * _A
_NSORT = 16384
_NPAD = 13008
_N_PRE = 12000
_NKEY = 12016
_N_POST = 600
_NPOST_PAD = 608
_NMS_IOU = 0.7
_MIN_SIZE = 16.0
_NEG_INF = float("-inf")


def _anchors_np():
    base_size = 16
    ratios = [0.5, 1, 2]
    anchor_scales = [8, 16, 32]
    ab = np.zeros((9, 4), dtype=np.float32)
    for i, r in enumerate(ratios):
        for j, s in enumerate(anchor_scales):
            h = base_size * s * np.sqrt(r)
            w = base_size * s * np.sqrt(1.0 / r)
            ab[i * 3 + j] = [-h / 2.0, -w / 2.0, h / 2.0, w / 2.0]
    sx = np.arange(0, _W * 16, 16)
    sy = np.arange(0, _H * 16, 16)
    sx, sy = np.meshgrid(sx, sy)
    shifts = np.vstack((sx.ravel(), sy.ravel(), sx.ravel(), sy.ravel())).transpose()
    anchors = ab.reshape((1, 9, 4)) + shifts.reshape((-1, 1, 4))
    return anchors.reshape((-1, 4)).astype(np.float32)


def _trunk_body(xt_ref, w_ref, cb_ref, wh_ref, bh_ref, out_ref, acc_ref):
    t = pl.program_id(0)

    @pl.when(t == 0)
    def _():
        acc_ref[...] = jnp.zeros_like(acc_ref)

    acc_ref[...] += jnp.dot(xt_ref[0], w_ref[0],
                            preferred_element_type=jnp.float32)

    @pl.when(t == 8)
    def _():
        feat = jnp.maximum(acc_ref[...] + cb_ref[...], 0.0)
        out_ref[...] = (jnp.dot(feat, wh_ref[...],
                                preferred_element_type=jnp.float32)
                        + bh_ref[...])


def _run_trunk(xt, w9, cb, wh, bh, interpret=False):
    m = _B * _K
    return pl.pallas_call(
        _trunk_body,
        grid=(9,),
        in_specs=[
            pl.BlockSpec((1, m, _CIN), lambda t: (t, 0, 0)),
            pl.BlockSpec((1, _CIN, _CMID), lambda t: (t, 0, 0)),
            pl.BlockSpec((1, _CMID), lambda t: (0, 0)),
            pl.BlockSpec((_CMID, 64), lambda t: (0, 0)),
            pl.BlockSpec((1, 64), lambda t: (0, 0)),
        ],
        out_specs=pl.BlockSpec((m, 64), lambda t: (0, 0)),
        out_shape=jax.ShapeDtypeStruct((m, 64), jnp.float32),
        scratch_shapes=[pltpu.VMEM((m, _CMID), jnp.float32)],
        interpret=interpret,
    )(xt, w9, cb, wh, bh)


def _bbox_body(a_ref, l_ref, s0_ref, s1_ref, img_ref, rois_ref, key_ref):
    m = img_ref[0]
    a0 = a_ref[0, :]
    a1 = a_ref[1, :]
    a2 = a_ref[2, :]
    a3 = a_ref[3, :]
    sw = a2 - a0
    sh = a3 - a1
    scx = a0 + 0.5 * sw
    scy = a1 + 0.5 * sh
    for b in range(_B):
        dx = l_ref[b, 0, :]
        dy = l_ref[b, 1, :]
        dw = l_ref[b, 2, :]
        dh = l_ref[b, 3, :]
        cx = dx * sw + scx
        cy = dy * sh + scy
        w = jnp.exp(dw) * sw
        h = jnp.exp(dh) * sh
        r0 = jnp.clip(cy - 0.5 * h, 0.0, m)
        r1 = jnp.clip(cx - 0.5 * w, 0.0, m)
        r2 = jnp.clip(cy + 0.5 * h, 0.0, m)
        r3 = jnp.clip(cx + 0.5 * w, 0.0, m)
        rois_ref[b, 0, :] = r0
        rois_ref[b, 1, :] = r1
        rois_ref[b, 2, :] = r2
        rois_ref[b, 3, :] = r3
        s0 = s0_ref[b, :]
        s1 = s1_ref[b, :]
        mx = jnp.maximum(s0, s1)
        e0 = jnp.exp(s0 - mx)
        e1 = jnp.exp(s1 - mx)
        p = e1 / (e0 + e1)
        valid = jnp.logical_and(r2 - r0 >= _MIN_SIZE, r3 - r1 >= _MIN_SIZE)
        key_ref[b, :] = jnp.where(valid, p, _NEG_INF)


def _run_bbox(a_t, locs_t, s0, s1, img, interpret=False):
    return pl.pallas_call(
        _bbox_body,
        in_specs=[
            pl.BlockSpec(memory_space=pltpu.VMEM),
            pl.BlockSpec(memory_space=pltpu.VMEM),
            pl.BlockSpec(memory_space=pltpu.VMEM),
            pl.BlockSpec(memory_space=pltpu.VMEM),
            pl.BlockSpec(memory_space=pltpu.SMEM),
        ],
        out_specs=[
            pl.BlockSpec(memory_space=pltpu.VMEM),
            pl.BlockSpec(memory_space=pltpu.VMEM),
        ],
        out_shape=[
            jax.ShapeDtypeStruct((_B, 4, _N), jnp.float32),
            jax.ShapeDtypeStruct((_B, _N), jnp.float32),
        ],
        interpret=interpret,
    )(a_t, locs_t, s0, s1, img)


def _xor_perm(x, j, axis):
    n = 128
    if axis == 1:
        left = jnp.concatenate([x[:, j:], x[:, :j]], axis=1)
        right = jnp.concatenate([x[:, n - j:], x[:, :n - j]], axis=1)
        sel = (lax.broadcasted_iota(jnp.int32, x.shape, 1) & j) == 0
    else:
        left = jnp.concatenate([x[j:, :], x[:j, :]], axis=0)
        right = jnp.concatenate([x[n - j:, :], x[:n - j, :]], axis=0)
        sel = (lax.broadcasted_iota(jnp.int32, x.shape, 0) & j) == 0
    return jnp.where(sel, left, right)


def _sort_body(key_ref, skey_ref, sidx_ref):
    kv = key_ref[0]
    r = lax.broadcasted_iota(jnp.int32, (128, 128), 0)
    c = lax.broadcasted_iota(jnp.int32, (128, 128), 1)
    i = r * 128 + c
    iv = i
    for kk_exp in range(1, 15):
        kk = 1 << kk_exp
        up = (i & kk) == 0
        for j_exp in range(kk_exp - 1, -1, -1):
            j = 1 << j_exp
            if j >= 128:
                ko = _xor_perm(kv, j // 128, axis=0)
                io = _xor_perm(iv, j // 128, axis=0)
            else:
                ko = _xor_perm(kv, j, axis=1)
                io = _xor_perm(iv, j, axis=1)
            lo = (i & j) == 0
            take_big = lo == up
            g = jnp.logical_or(kv > ko,
                               jnp.logical_and(kv == ko, iv < io))
            mtake = g == take_big
            kv = jnp.where(mtake, kv, ko)
            iv = jnp.where(mtake, iv, io)
    skey_ref[0] = kv
    sidx_ref[0] = iv


def _run_sort(keyp, interpret=False):
    return pl.pallas_call(
        _sort_body,
        grid=(_B,),
        in_specs=[pl.BlockSpec((1, 128, 128), lambda b: (b, 0, 0))],
        out_specs=[
            pl.BlockSpec((1, 128, 128), lambda b: (b, 0, 0)),
            pl.BlockSpec((1, 128, 128), lambda b: (b, 0, 0)),
        ],
        out_shape=[
            jax.ShapeDtypeStruct((_B, 128, 128), jnp.float32),
            jax.ShapeDtypeStruct((_B, 128, 128), jnp.int32),
        ],
        interpret=interpret,
    )(keyp)


def _nms_body(rois_hbm, sidx_hbm, skey_hbm, out_hbm,
              y1_v, x1_v, y2_v, x2_v, sidx_v, skey_v,
              ky1_v, kx1_v, ky2_v, kx2_v, karea_v):
    cid = lax.axis_index("c")
    sid = lax.axis_index("s")

    @pl.when(sid == 0)
    def _work():
        w = cid
        pltpu.sync_copy(rois_hbm.at[w, 0], y1_v)
        pltpu.sync_copy(rois_hbm.at[w, 1], x1_v)
        pltpu.sync_copy(rois_hbm.at[w, 2], y2_v)
        pltpu.sync_copy(rois_hbm.at[w, 3], x2_v)
        pltpu.sync_copy(sidx_hbm.at[w, pl.ds(0, _N_PRE)], sidx_v)
        pltpu.sync_copy(skey_hbm.at[w, pl.ds(0, _NKEY)], skey_v)

        def outer_cond(st):
            ptr, cnt = st
            in_range = jnp.logical_and(ptr < _N_PRE, cnt < _N_POST)
            return jnp.logical_and(in_range, skey_v[ptr] > _NEG_INF)

        def outer_body(st):
            ptr, cnt = st
            gid = sidx_v[ptr]
            cy1 = y1_v[gid]
            cx1 = x1_v[gid]
            cy2 = y2_v[gid]
            cx2 = x2_v[gid]
            ca = (cy2 - cy1) * (cx2 - cx1)
            nch = (cnt + 15) // 16

            def chk_cond(s2):
                cc, hit = s2
                return jnp.logical_and(cc < nch, jnp.logical_not(hit))

            def chk_body(s2):
                cc, hit = s2
                sl = pl.ds(cc * 16, 16)
                k1 = ky1_v[sl]
                k2 = kx1_v[sl]
                k3 = ky2_v[sl]
                k4 = kx2_v[sl]
                ka = karea_v[sl]
                yy1 = jnp.maximum(cy1, k1)
                xx1 = jnp.maximum(cx1, k2)
                yy2 = jnp.minimum(cy2, k3)
                xx2 = jnp.minimum(cx2, k4)
                inter = (jnp.maximum(0.0, yy2 - yy1)
                         * jnp.maximum(0.0, xx2 - xx1))
                iou = inter / (ka + ca - inter + 1e-12)
                lane = lax.iota(jnp.int32, 16) + cc * 16
                anyhit = jnp.any(jnp.logical_and(iou > _NMS_IOU, lane < cnt))
                return cc + 1, jnp.logical_or(hit, anyhit)

            _, suppressed = lax.while_loop(
                chk_cond, chk_body, (jnp.int32(0), jnp.bool_(False)))

            @pl.when(jnp.logical_not(suppressed))
            def _take():
                ky1_v[cnt] = cy1
                kx1_v[cnt] = cx1
                ky2_v[cnt] = cy2
                kx2_v[cnt] = cx2
                karea_v[cnt] = ca

            cnt2 = jnp.where(suppressed, cnt, cnt + 1)
            return ptr + 1, cnt2

        ptr, cnt = lax.while_loop(outer_cond, outer_body,
                                  (jnp.int32(0), jnp.int32(0)))

        li = jnp.maximum(cnt - 1, 0)
        has = cnt > 0
        ly1 = jnp.where(has, ky1_v[li], y1_v[0])
        lx1 = jnp.where(has, kx1_v[li], x1_v[0])
        ly2 = jnp.where(has, ky2_v[li], y2_v[0])
        lx2 = jnp.where(has, kx2_v[li], x2_v[0])
        for ccst in range(_NPOST_PAD // 16):
            sl = pl.ds(ccst * 16, 16)
            lane = lax.iota(jnp.int32, 16) + ccst * 16
            keepmask = lane < cnt
            ky1_v[sl] = jnp.where(keepmask, ky1_v[sl], ly1)
            kx1_v[sl] = jnp.where(keepmask, kx1_v[sl], lx1)
            ky2_v[sl] = jnp.where(keepmask, ky2_v[sl], ly2)
            kx2_v[sl] = jnp.where(keepmask, kx2_v[sl], lx2)
        pltpu.sync_copy(ky1_v, out_hbm.at[w, 0])
        pltpu.sync_copy(kx1_v, out_hbm.at[w, 1])
        pltpu.sync_copy(ky2_v, out_hbm.at[w, 2])
        pltpu.sync_copy(kx2_v, out_hbm.at[w, 3])


def _run_nms(rois_p, sidx, skey):
    mesh = plsc.VectorSubcoreMesh(core_axis_name="c", subcore_axis_name="s")
    kd = functools.partial(
        pl.kernel,
        out_type=jax.ShapeDtypeStruct((_B, 4, _NPOST_PAD), jnp.float32),
        mesh=mesh,
        scratch_types=[
            pltpu.VMEM((_NPAD,), jnp.float32),
            pltpu.VMEM((_NPAD,), jnp.float32),
            pltpu.VMEM((_NPAD,), jnp.float32),
            pltpu.VMEM((_NPAD,), jnp.float32),
            pltpu.VMEM((_N_PRE,), jnp.int32),
            pltpu.VMEM((_NKEY,), jnp.float32),
            pltpu.VMEM((_NPOST_PAD,), jnp.float32),
            pltpu.VMEM((_NPOST_PAD,), jnp.float32),
            pltpu.VMEM((_NPOST_PAD,), jnp.float32),
            pltpu.VMEM((_NPOST_PAD,), jnp.float32),
            pltpu.VMEM((_NPOST_PAD,), jnp.float32),
        ],
    )(_nms_body)
    return kd(rois_p, sidx, skey)


def kernel(x, img_size, conv_w, conv_b, score_w, score_b, loc_w, loc_b):
    xn = jnp.transpose(x, (0, 2, 3, 1))
    xp = jnp.pad(xn, ((0, 0), (1, 1), (1, 1), (0, 0)))
    xt = jnp.stack([
        xp[:, kh:kh + _H, kw:kw + _W, :].reshape(_B * _K, _CIN)
        for kh in range(3) for kw in range(3)
    ])
    w9 = jnp.transpose(conv_w, (2, 3, 1, 0)).reshape(9, _CIN, _CMID)
    cb = conv_b.reshape(1, _CMID)
    wl = jnp.transpose(loc_w.reshape(_A * 4, _CMID), (1, 0))
    ws = jnp.transpose(score_w.reshape(_A * 2, _CMID), (1, 0))
    wh = jnp.concatenate(
        [wl, ws, jnp.zeros((_CMID, 64 - _A * 6), jnp.float32)], axis=1)
    bh = jnp.concatenate(
        [loc_b, score_b, jnp.zeros((64 - _A * 6,), jnp.float32)]).reshape(1, 64)

    out2 = _run_trunk(xt, w9, cb, wh, bh)
    rpn_locs = out2[:, :_A * 4].reshape(_B, _N, 4)
    rpn_scores = out2[:, _A * 4:_A * 6].reshape(_B, _N, 2)

    anchors = jnp.asarray(_anchors_np())
    a_t = jnp.transpose(anchors, (1, 0))
    locs_t = jnp.transpose(rpn_locs, (0, 2, 1))
    s0 = rpn_scores[:, :, 0]
    s1 = rpn_scores[:, :, 1]
    img = jnp.asarray(img_size, jnp.float32).reshape(1)

    rois_t, key = _run_bbox(a_t, locs_t, s0, s1, img)

    keyp = jnp.concatenate(
        [key, jnp.full((_B, _NSORT - _N), _NEG_INF, jnp.float32)],
        axis=1).reshape(_B, 128, 128)
    skey3, sidx3 = _run_sort(keyp)
    skey = skey3.reshape(_B, _NSORT)
    sidx = sidx3.reshape(_B, _NSORT)

    rois_p = jnp.pad(rois_t, ((0, 0), (0, 0), (0, _NPAD - _N)))
    out_t = _run_nms(rois_p, sidx, skey)
    rois = jnp.transpose(out_t[:, :, :_N_POST], (0, 2, 1))

    roi_indices = jnp.broadcast_to(
        jnp.arange(_B, dtype=jnp.float32)[:, None], (_B, _N_POST))
    return rpn_locs, rpn_scores, rois, roi_indices, anchors


# speedup vs baseline: 100.6819x; 100.6819x over previous
import functools

import numpy as np
import jax
import jax.numpy as jnp
from jax import lax
from jax.experimental import pallas as pl
from jax.experimental.pallas import tpu as pltpu
from jax.experimental.pallas import tpu_sc as plsc

_B, _H, _W = 2, 38, 38
_CIN = 512
_CMID = 512
_A = 9
_K = _H * _W
_N = _K * _A
_NSORT = 16384
_NPAD = 13024
_N_PRE = 12000
_NKEY = 12016
_N_POST = 600
_NPOST_PAD = 608
_NKEEP = 624
_NMS_IOU = 0.7
_MIN_SIZE = 16.0
_NEG_INF = float("-inf")


def _anchors_np():
    base_size = 16
    ratios = [0.5, 1, 2]
    anchor_scales = [8, 16, 32]
    ab = np.zeros((9, 4), dtype=np.float32)
    for i, r in enumerate(ratios):
        for j, s in enumerate(anchor_scales):
            h = base_size * s * np.sqrt(r)
            w = base_size * s * np.sqrt(1.0 / r)
            ab[i * 3 + j] = [-h / 2.0, -w / 2.0, h / 2.0, w / 2.0]
    sx = np.arange(0, _W * 16, 16)
    sy = np.arange(0, _H * 16, 16)
    sx, sy = np.meshgrid(sx, sy)
    shifts = np.vstack((sx.ravel(), sy.ravel(), sx.ravel(), sy.ravel())).transpose()
    anchors = ab.reshape((1, 9, 4)) + shifts.reshape((-1, 1, 4))
    return anchors.reshape((-1, 4)).astype(np.float32)


def _trunk_body(xt_ref, w_ref, cb_ref, wh_ref, bh_ref, out_ref, acc_ref):
    t = pl.program_id(0)

    @pl.when(t == 0)
    def _():
        acc_ref[...] = jnp.zeros_like(acc_ref)

    acc_ref[...] += jnp.dot(xt_ref[0], w_ref[0],
                            preferred_element_type=jnp.float32)

    @pl.when(t == 8)
    def _():
        feat = jnp.maximum(acc_ref[...] + cb_ref[...], 0.0)
        out_ref[...] = (jnp.dot(feat, wh_ref[...],
                                preferred_element_type=jnp.float32)
                        + bh_ref[...])


def _run_trunk(xt, w9, cb, wh, bh, interpret=False):
    m = _B * _K
    return pl.pallas_call(
        _trunk_body,
        grid=(9,),
        in_specs=[
            pl.BlockSpec((1, m, _CIN), lambda t: (t, 0, 0)),
            pl.BlockSpec((1, _CIN, _CMID), lambda t: (t, 0, 0)),
            pl.BlockSpec((1, _CMID), lambda t: (0, 0)),
            pl.BlockSpec((_CMID, 64), lambda t: (0, 0)),
            pl.BlockSpec((1, 64), lambda t: (0, 0)),
        ],
        out_specs=pl.BlockSpec((m, 64), lambda t: (0, 0)),
        out_shape=jax.ShapeDtypeStruct((m, 64), jnp.float32),
        scratch_shapes=[pltpu.VMEM((m, _CMID), jnp.float32)],
        interpret=interpret,
    )(xt, w9, cb, wh, bh)


def _bbox_body(a_ref, l_ref, s0_ref, s1_ref, img_ref, rois_ref, key_ref):
    m = img_ref[0]
    a0 = a_ref[0, :]
    a1 = a_ref[1, :]
    a2 = a_ref[2, :]
    a3 = a_ref[3, :]
    sw = a2 - a0
    sh = a3 - a1
    scx = a0 + 0.5 * sw
    scy = a1 + 0.5 * sh
    for b in range(_B):
        dx = l_ref[b, 0, :]
        dy = l_ref[b, 1, :]
        dw = l_ref[b, 2, :]
        dh = l_ref[b, 3, :]
        cx = dx * sw + scx
        cy = dy * sh + scy
        w = jnp.exp(dw) * sw
        h = jnp.exp(dh) * sh
        r0 = jnp.clip(cy - 0.5 * h, 0.0, m)
        r1 = jnp.clip(cx - 0.5 * w, 0.0, m)
        r2 = jnp.clip(cy + 0.5 * h, 0.0, m)
        r3 = jnp.clip(cx + 0.5 * w, 0.0, m)
        rois_ref[b, 0, :] = r0
        rois_ref[b, 1, :] = r1
        rois_ref[b, 2, :] = r2
        rois_ref[b, 3, :] = r3
        s0 = s0_ref[b, :]
        s1 = s1_ref[b, :]
        mx = jnp.maximum(s0, s1)
        e0 = jnp.exp(s0 - mx)
        e1 = jnp.exp(s1 - mx)
        p = e1 / (e0 + e1)
        valid = jnp.logical_and(r2 - r0 >= _MIN_SIZE, r3 - r1 >= _MIN_SIZE)
        key_ref[b, :] = jnp.where(valid, p, _NEG_INF)


def _run_bbox(a_t, locs_t, s0, s1, img, interpret=False):
    return pl.pallas_call(
        _bbox_body,
        in_specs=[
            pl.BlockSpec(memory_space=pltpu.VMEM),
            pl.BlockSpec(memory_space=pltpu.VMEM),
            pl.BlockSpec(memory_space=pltpu.VMEM),
            pl.BlockSpec(memory_space=pltpu.VMEM),
            pl.BlockSpec(memory_space=pltpu.SMEM),
        ],
        out_specs=[
            pl.BlockSpec(memory_space=pltpu.VMEM),
            pl.BlockSpec(memory_space=pltpu.VMEM),
        ],
        out_shape=[
            jax.ShapeDtypeStruct((_B, 4, _N), jnp.float32),
            jax.ShapeDtypeStruct((_B, _N), jnp.float32),
        ],
        interpret=interpret,
    )(a_t, locs_t, s0, s1, img)


def _xor_perm(x, j, axis):
    n = 128
    if axis == 1:
        left = jnp.concatenate([x[:, j:], x[:, :j]], axis=1)
        right = jnp.concatenate([x[:, n - j:], x[:, :n - j]], axis=1)
        sel = (lax.broadcasted_iota(jnp.int32, x.shape, 1) & j) == 0
    else:
        left = jnp.concatenate([x[j:, :], x[:j, :]], axis=0)
        right = jnp.concatenate([x[n - j:, :], x[:n - j, :]], axis=0)
        sel = (lax.broadcasted_iota(jnp.int32, x.shape, 0) & j) == 0
    return jnp.where(sel, left, right)


def _sort_body(key_ref, skey_ref, sidx_ref):
    kv = key_ref[0]
    r = lax.broadcasted_iota(jnp.int32, (128, 128), 0)
    c = lax.broadcasted_iota(jnp.int32, (128, 128), 1)
    i = r * 128 + c
    iv = i
    for kk_exp in range(1, 15):
        kk = 1 << kk_exp
        up = (i & kk) == 0
        for j_exp in range(kk_exp - 1, -1, -1):
            j = 1 << j_exp
            if j >= 128:
                ko = _xor_perm(kv, j // 128, axis=0)
                io = _xor_perm(iv, j // 128, axis=0)
            else:
                ko = _xor_perm(kv, j, axis=1)
                io = _xor_perm(iv, j, axis=1)
            lo = (i & j) == 0
            take_big = lo == up
            g = jnp.logical_or(kv > ko,
                               jnp.logical_and(kv == ko, iv < io))
            mtake = g == take_big
            kv = jnp.where(mtake, kv, ko)
            iv = jnp.where(mtake, iv, io)
    skey_ref[0] = kv
    sidx_ref[0] = iv


def _run_sort(keyp, interpret=False):
    return pl.pallas_call(
        _sort_body,
        grid=(_B,),
        in_specs=[pl.BlockSpec((1, 128, 128), lambda b: (b, 0, 0))],
        out_specs=[
            pl.BlockSpec((1, 128, 128), lambda b: (b, 0, 0)),
            pl.BlockSpec((1, 128, 128), lambda b: (b, 0, 0)),
        ],
        out_shape=[
            jax.ShapeDtypeStruct((_B, 128, 128), jnp.float32),
            jax.ShapeDtypeStruct((_B, 128, 128), jnp.int32),
        ],
        interpret=interpret,
    )(keyp)


def _sload(ref, i):
    return ref[pl.ds(i, 16)][0]


def _sstore(ref, i, val):
    ref[pl.ds(i, 16)] = jnp.full((16,), val, ref.dtype)


def _nms_body(rois_hbm, sidx_hbm, skey_hbm, out_hbm,
              y1_v, x1_v, y2_v, x2_v, sidx_v, skey_v,
              ky1_v, kx1_v, ky2_v, kx2_v, karea_v):
    cid = lax.axis_index("c")
    sid = lax.axis_index("s")

    w = cid
    rbase = pl.multiple_of(w * (4 * _NPAD), 8)
    pltpu.sync_copy(rois_hbm.at[pl.ds(rbase, _NPAD)], y1_v)
    pltpu.sync_copy(rois_hbm.at[pl.ds(rbase + _NPAD, _NPAD)], x1_v)
    pltpu.sync_copy(rois_hbm.at[pl.ds(rbase + 2 * _NPAD, _NPAD)], y2_v)
    pltpu.sync_copy(rois_hbm.at[pl.ds(rbase + 3 * _NPAD, _NPAD)], x2_v)
    sbase = pl.multiple_of(w * _NSORT, 8)
    pltpu.sync_copy(sidx_hbm.at[pl.ds(sbase, _NKEY)], sidx_v)
    pltpu.sync_copy(skey_hbm.at[pl.ds(sbase, _NKEY)], skey_v)

    def outer_body(ptr, st):
        cnt, stop = st
        live = jnp.logical_and(
            jnp.logical_and(_sload(skey_v, ptr) > _NEG_INF,
                            jnp.logical_not(stop)),
            cnt < _N_POST)
        gid = _sload(sidx_v, ptr)
        cy1 = _sload(y1_v, gid)
        cx1 = _sload(x1_v, gid)
        cy2 = _sload(y2_v, gid)
        cx2 = _sload(x2_v, gid)
        ca = (cy2 - cy1) * (cx2 - cx1)
        nch = jnp.where(live, (cnt + 15) // 16, 0)

        def chk_body(cc, hitv):
            sl = pl.ds(cc * 16, 16)
            k1 = ky1_v[sl]
            k2 = kx1_v[sl]
            k3 = ky2_v[sl]
            k4 = kx2_v[sl]
            ka = karea_v[sl]
            yy1 = jnp.maximum(cy1, k1)
            xx1 = jnp.maximum(cx1, k2)
            yy2 = jnp.minimum(cy2, k3)
            xx2 = jnp.minimum(cx2, k4)
            inter = (jnp.maximum(0.0, yy2 - yy1)
                     * jnp.maximum(0.0, xx2 - xx1))
            iou = inter / (ka + ca - inter + 1e-12)
            lane = lax.iota(jnp.int32, 16) + cc * 16
            a = jnp.where(iou > _NMS_IOU, 1.0, 0.0)
            b = jnp.where(lane < cnt, 1.0, 0.0)
            return jnp.maximum(hitv, a * b)

        hitv = lax.fori_loop(0, nch, chk_body,
                             jnp.zeros((16,), jnp.float32))
        suppressed = plsc.all_reduce_population_count(hitv > 0.5)[0] > 0

        take = jnp.logical_and(live, jnp.logical_not(suppressed))
        dst = jnp.where(take, cnt, _NPOST_PAD)
        _sstore(ky1_v, dst, cy1)
        _sstore(kx1_v, dst, cx1)
        _sstore(ky2_v, dst, cy2)
        _sstore(kx2_v, dst, cx2)
        _sstore(karea_v, dst, ca)

        cnt2 = jnp.where(take, cnt + 1, cnt)
        stop2 = jnp.logical_or(stop, _sload(skey_v, ptr) <= _NEG_INF)
        return cnt2, stop2

    cnt, _ = lax.fori_loop(0, _N_PRE, outer_body,
                           (jnp.int32(0), jnp.bool_(False)))

    li = jnp.maximum(cnt - 1, 0)
    has = cnt > 0
    ly1 = jnp.where(has, _sload(ky1_v, li), _sload(y1_v, 0))
    lx1 = jnp.where(has, _sload(kx1_v, li), _sload(x1_v, 0))
    ly2 = jnp.where(has, _sload(ky2_v, li), _sload(y2_v, 0))
    lx2 = jnp.where(has, _sload(kx2_v, li), _sload(x2_v, 0))
    for ccst in range(_NPOST_PAD // 16):
        sl = pl.ds(ccst * 16, 16)
        lane = lax.iota(jnp.int32, 16) + ccst * 16
        keepmask = lane < cnt
        ky1_v[sl] = jnp.where(keepmask, ky1_v[sl], ly1)
        kx1_v[sl] = jnp.where(keepmask, kx1_v[sl], lx1)
        ky2_v[sl] = jnp.where(keepmask, ky2_v[sl], ly2)
        kx2_v[sl] = jnp.where(keepmask, kx2_v[sl], lx2)

    @pl.when(sid == 0)
    def _writeout():
        obase = pl.multiple_of(w * (4 * _NPOST_PAD), 8)
        pltpu.sync_copy(ky1_v.at[pl.ds(0, _NPOST_PAD)],
                        out_hbm.at[pl.ds(obase, _NPOST_PAD)])
        pltpu.sync_copy(kx1_v.at[pl.ds(0, _NPOST_PAD)],
                        out_hbm.at[pl.ds(obase + _NPOST_PAD, _NPOST_PAD)])
        pltpu.sync_copy(ky2_v.at[pl.ds(0, _NPOST_PAD)],
                        out_hbm.at[pl.ds(obase + 2 * _NPOST_PAD, _NPOST_PAD)])
        pltpu.sync_copy(kx2_v.at[pl.ds(0, _NPOST_PAD)],
                        out_hbm.at[pl.ds(obase + 3 * _NPOST_PAD, _NPOST_PAD)])


def _run_nms(rois_p, sidx, skey):
    mesh = plsc.VectorSubcoreMesh(core_axis_name="c", subcore_axis_name="s")
    kd = functools.partial(
        pl.kernel,
        out_type=jax.ShapeDtypeStruct((_B * 4 * _NPOST_PAD,), jnp.float32),
        mesh=mesh,
        compiler_params=pltpu.CompilerParams(needs_layout_passes=False),
        scratch_types=[
            pltpu.VMEM((_NPAD,), jnp.float32),
            pltpu.VMEM((_NPAD,), jnp.float32),
            pltpu.VMEM((_NPAD,), jnp.float32),
            pltpu.VMEM((_NPAD,), jnp.float32),
            pltpu.VMEM((_NKEY,), jnp.int32),
            pltpu.VMEM((_NKEY,), jnp.float32),
            pltpu.VMEM((_NKEEP,), jnp.float32),
            pltpu.VMEM((_NKEEP,), jnp.float32),
            pltpu.VMEM((_NKEEP,), jnp.float32),
            pltpu.VMEM((_NKEEP,), jnp.float32),
            pltpu.VMEM((_NKEEP,), jnp.float32),
        ],
    )(_nms_body)
    return kd(rois_p.reshape(-1), sidx.reshape(-1),
              skey.reshape(-1)).reshape(_B, 4, _NPOST_PAD)


def kernel(x, img_size, conv_w, conv_b, score_w, score_b, loc_w, loc_b):
    xn = jnp.transpose(x, (0, 2, 3, 1))
    xp = jnp.pad(xn, ((0, 0), (1, 1), (1, 1), (0, 0)))
    xt = jnp.stack([
        xp[:, kh:kh + _H, kw:kw + _W, :].reshape(_B * _K, _CIN)
        for kh in range(3) for kw in range(3)
    ])
    w9 = jnp.transpose(conv_w, (2, 3, 1, 0)).reshape(9, _CIN, _CMID)
    cb = conv_b.reshape(1, _CMID)
    wl = jnp.transpose(loc_w.reshape(_A * 4, _CMID), (1, 0))
    ws = jnp.transpose(score_w.reshape(_A * 2, _CMID), (1, 0))
    wh = jnp.concatenate(
        [wl, ws, jnp.zeros((_CMID, 64 - _A * 6), jnp.float32)], axis=1)
    bh = jnp.concatenate(
        [loc_b, score_b, jnp.zeros((64 - _A * 6,), jnp.float32)]).reshape(1, 64)

    out2 = _run_trunk(xt, w9, cb, wh, bh)
    rpn_locs = out2[:, :_A * 4].reshape(_B, _N, 4)
    rpn_scores = out2[:, _A * 4:_A * 6].reshape(_B, _N, 2)

    anchors = jnp.asarray(_anchors_np())
    a_t = jnp.transpose(anchors, (1, 0))
    locs_t = jnp.transpose(rpn_locs, (0, 2, 1))
    s0 = rpn_scores[:, :, 0]
    s1 = rpn_scores[:, :, 1]
    img = jnp.asarray(img_size, jnp.float32).reshape(1)

    rois_t, key = _run_bbox(a_t, locs_t, s0, s1, img)

    keyp = jnp.concatenate(
        [key, jnp.full((_B, _NSORT - _N), _NEG_INF, jnp.float32)],
        axis=1).reshape(_B, 128, 128)
    skey3, sidx3 = _run_sort(keyp)
    skey = skey3.reshape(_B, _NSORT)
    sidx = sidx3.reshape(_B, _NSORT)

    rois_p = jnp.pad(rois_t, ((0, 0), (0, 0), (0, _NPAD - _N)))
    out_t = _run_nms(rois_p, sidx, skey)
    rois = jnp.transpose(out_t[:, :, :_N_POST], (0, 2, 1))

    roi_indices = jnp.broadcast_to(
        jnp.arange(_B, dtype=jnp.float32)[:, None], (_B, _N_POST))
    return rpn_locs, rpn_scores, rois, roi_indices, anchors
